# Initial kernel scaffold; baseline (speedup 1.0000x reference)
import jax
import jax.numpy as jnp
from jax.experimental import pallas as pl


def kernel(feat, edge_index, Wl1, bl1, Wr1, Wl2, bl2, Wr2, Wl3, bl3, Wr3, Wfc, bfc):
    raise NotImplementedError("write your pallas kernel here")


# speedup vs baseline: 2.7354x; 2.7354x over previous
import functools

import jax
import jax.numpy as jnp
from jax import lax
from jax.experimental import pallas as pl
from jax.experimental.pallas import tpu as pltpu
from jax.experimental.pallas import tpu_sc as plsc

NC = 2
NS = 16
NW = NC * NS
CH = 128
CNTW = 16


def _sc_mesh():
  return plsc.VectorSubcoreMesh(
      core_axis_name="c", subcore_axis_name="s", num_cores=NC,
      num_subcores=NS)


def _make_sc_agg(n_pad, d, cpt):
  rpt = n_pad // NS

  scratch = [
      pltpu.VMEM_SHARED((n_pad, d), jnp.float32),
      pltpu.VMEM((cpt, CH), jnp.int32),
      pltpu.VMEM((2, CH), jnp.int32),
      pltpu.VMEM((CH, d), jnp.float32),
      pltpu.VMEM((CH, d), jnp.float32),
      pltpu.SemaphoreType.DMA,
      pltpu.SemaphoreType.DMA,
      pltpu.SemaphoreType.DMA,
      pltpu.SemaphoreType.DMA,
  ]

  def body(table, srcs, dsts, zeros, out,
           acc, dst_v, src_ib, rb0, rb1, sem0, sem1, semi0, semi1):
    c = lax.axis_index("c")
    s = lax.axis_index("s")
    wid = c * NS + s

    pltpu.sync_copy(zeros.at[pl.ds(s * rpt, rpt)], acc.at[pl.ds(s * rpt, rpt)])
    pltpu.sync_copy(dsts.at[wid], dst_v)
    plsc.subcore_barrier()

    rbufs = (rb0, rb1)
    sems = (sem0, sem1)
    isems = (semi0, semi1)

    pltpu.sync_copy(srcs.at[wid, 0], src_ib.at[0])
    pltpu.sync_copy(srcs.at[wid, 1], src_ib.at[1])
    pltpu.async_copy(table.at[src_ib.at[0]], rb0, sem0)
    pltpu.async_copy(table.at[src_ib.at[1]], rb1, sem1)

    def step(j, b, fetch_next):
      pltpu.make_async_copy(table.at[src_ib.at[b]], rbufs[b], sems[b]).wait()
      if fetch_next:
        pltpu.async_copy(srcs.at[wid, j + 2], src_ib.at[b], isems[b])
      pltpu.sync_copy(rbufs[b], acc.at[dst_v.at[j]], add=True)
      if fetch_next:
        pltpu.make_async_copy(srcs.at[wid, j + 2], src_ib.at[b],
                              isems[b]).wait()
        pltpu.async_copy(table.at[src_ib.at[b]], rbufs[b], sems[b])

    def loop_body(g, _):
      for b in range(2):
        step(g * 2 + b, b, True)
      return _

    lax.fori_loop(0, cpt // 2 - 1, loop_body, None)
    step(cpt - 2, 0, False)
    step(cpt - 1, 1, False)

    plsc.subcore_barrier()
    pltpu.sync_copy(acc.at[pl.ds(s * rpt, rpt)],
                    out.at[c, pl.ds(s * rpt, rpt)])

  return pl.kernel(
      body,
      out_type=jax.ShapeDtypeStruct((NC, n_pad, d), jnp.float32),
      mesh=_sc_mesh(),
      scratch_types=scratch)


def _make_sc_cnt(n_pad, cpt):
  rpt = n_pad // NS

  scratch = [
      pltpu.VMEM_SHARED((n_pad, CNTW), jnp.float32),
      pltpu.VMEM((cpt, CH), jnp.int32),
      pltpu.VMEM((CH, CNTW), jnp.float32),
  ]

  def body(dsts, zeros_c, ones, cnt_out, cnt_acc, dst_v, ones_v):
    c = lax.axis_index("c")
    s = lax.axis_index("s")
    wid = c * NS + s

    pltpu.sync_copy(zeros_c.at[pl.ds(s * rpt, rpt)],
                    cnt_acc.at[pl.ds(s * rpt, rpt)])
    pltpu.sync_copy(dsts.at[wid], dst_v)
    pltpu.sync_copy(ones, ones_v)
    plsc.subcore_barrier()

    def loop_body(j, _):
      pltpu.sync_copy(ones_v, cnt_acc.at[dst_v.at[j]], add=True)
      return _

    lax.fori_loop(0, cpt, loop_body, None)

    plsc.subcore_barrier()
    pltpu.sync_copy(cnt_acc.at[pl.ds(s * rpt, rpt)],
                    cnt_out.at[c, pl.ds(s * rpt, rpt)])

  return pl.kernel(
      body,
      out_type=jax.ShapeDtypeStruct((NC, n_pad, CNTW), jnp.float32),
      mesh=_sc_mesh(),
      scratch_types=scratch)


def _sage_dense_body(a0, a1, c0, c1, x, wl, bl, wr, o, *, relu):
  cnt = jnp.maximum(c0[...] + c1[...], 1.0)
  agg = (a0[...] + a1[...]) / cnt
  h = (jnp.dot(agg, wl[...], preferred_element_type=jnp.float32) + bl[...]
       + jnp.dot(x[...], wr[...], preferred_element_type=jnp.float32))
  nrm = jnp.sqrt(jnp.sum(h * h, axis=-1, keepdims=True))
  h = h / jnp.maximum(nrm, 1e-12)
  if relu:
    h = jnp.maximum(h, 0.0)
  o[...] = h


def _final_body(a0, a1, c0, c1, x, wl, bl, wr, wfc, bfc, o):
  cnt = jnp.maximum(c0[...] + c1[...], 1.0)
  agg = (a0[...] + a1[...]) / cnt
  h = (jnp.dot(agg, wl[...], preferred_element_type=jnp.float32) + bl[...]
       + jnp.dot(x[...], wr[...], preferred_element_type=jnp.float32))
  nrm = jnp.sqrt(jnp.sum(h * h, axis=-1, keepdims=True))
  h = h / jnp.maximum(nrm, 1e-12)
  logits = jnp.dot(h, wfc[...], preferred_element_type=jnp.float32) + bfc[...]
  m = jnp.max(logits, axis=-1, keepdims=True)
  e = jnp.exp(logits - m)
  o[...] = e / jnp.sum(e, axis=-1, keepdims=True)


def _dense_call(body, n, d, br, extra_w):
  grid = (n // br,)
  row_spec = pl.BlockSpec((br, d), lambda i: (i, 0))
  cnt_spec = pl.BlockSpec((br, 1), lambda i: (i, 0))
  w_spec = pl.BlockSpec((d, d), lambda i: (0, 0))
  b_spec = pl.BlockSpec((1, d), lambda i: (0, 0))
  in_specs = [row_spec, row_spec, cnt_spec, cnt_spec, row_spec,
              w_spec, b_spec, w_spec] + [w_spec, b_spec] * extra_w
  return pl.pallas_call(
      body,
      grid=grid,
      in_specs=in_specs,
      out_specs=pl.BlockSpec((br, d), lambda i: (i, 0)),
      out_shape=jax.ShapeDtypeStruct((n, d), jnp.float32),
  )


def kernel(feat, edge_index, Wl1, bl1, Wr1, Wl2, bl2, Wr2, Wl3, bl3, Wr3,
           Wfc, bfc):
  n, d = feat.shape
  e = edge_index.shape[1]
  out_dim = Wfc.shape[0]

  cpt = -(-e // (NW * CH))
  if cpt % 2:
    cpt += 1
  cap = NW * cpt * CH
  n_pad = -(-(n + 1) // (NS * 8)) * (NS * 8)

  src = edge_index[0]
  dst = edge_index[1]
  pad = cap - e
  src_p = jnp.concatenate(
      [src, jnp.zeros((pad,), jnp.int32)]).reshape(NW, cpt, CH)
  dst_p = jnp.concatenate(
      [dst, jnp.full((pad,), n, jnp.int32)]).reshape(NW, cpt, CH)

  zeros = jnp.zeros((n_pad, d), jnp.float32)

  agg = _make_sc_agg(n_pad, d, cpt)

  br = 1000 if n % 1000 == 0 else n
  dense = _dense_call(functools.partial(_sage_dense_body, relu=True),
                      n, d, br, 0)
  final = _dense_call(_final_body, n, d, br, 1)

  c_parts = agg(jnp.ones((n, d), jnp.float32), src_p, dst_p, zeros)
  c0 = c_parts[0, :n, 0:1]
  c1 = c_parts[1, :n, 0:1]

  a_parts = agg(feat, src_p, dst_p, zeros)
  h1 = dense(a_parts[0, :n], a_parts[1, :n], c0, c1, feat,
             Wl1.T, bl1[None, :], Wr1.T)
  a_parts2 = agg(h1, src_p, dst_p, zeros)
  h2 = dense(a_parts2[0, :n], a_parts2[1, :n], c0, c1, h1,
             Wl2.T, bl2[None, :], Wr2.T)
  a_parts3 = agg(h2, src_p, dst_p, zeros)

  wfc_pad = jnp.zeros((d, d), jnp.float32).at[:, :out_dim].set(Wfc.T)
  bfc_pad = jnp.full((1, d), -1e30, jnp.float32).at[0, :out_dim].set(bfc)
  out = final(a_parts3[0, :n], a_parts3[1, :n], c0, c1, h2,
              Wl3.T, bl3[None, :], Wr3.T, wfc_pad, bfc_pad)
  return out[:, :out_dim]


# speedup vs baseline: 2.7407x; 1.0019x over previous
import functools

import jax
import jax.numpy as jnp
from jax import lax
from jax.experimental import pallas as pl
from jax.experimental.pallas import tpu as pltpu
from jax.experimental.pallas import tpu_sc as plsc

NC = 2
NS = 16
NW = NC * NS


def _sc_mesh():
  return plsc.VectorSubcoreMesh(
      core_axis_name="c", subcore_axis_name="s", num_cores=NC,
      num_subcores=NS)


def _make_sc_agg(n_pad, d, cpt, ch, nbuf):
  rpt = n_pad // NS

  scratch = (
      [pltpu.VMEM_SHARED((n_pad, d), jnp.float32)] +
      [pltpu.VMEM((cpt, ch), jnp.int32)] +
      [pltpu.VMEM((nbuf, ch), jnp.int32)] +
      [pltpu.VMEM((ch, d), jnp.float32)] * nbuf +
      [pltpu.SemaphoreType.DMA] * (2 * nbuf))

  def body(table, srcs, dsts, zeros, out, acc, src_v, dst_ib, *rest):
    rbufs = rest[:nbuf]
    gsems = rest[nbuf:2 * nbuf]
    dsems = rest[2 * nbuf:]

    c = lax.axis_index("c")
    s = lax.axis_index("s")
    wid = c * NS + s

    pltpu.sync_copy(zeros.at[pl.ds(s * rpt, rpt)], acc.at[pl.ds(s * rpt, rpt)])
    pltpu.sync_copy(srcs.at[wid], src_v)
    plsc.subcore_barrier()

    def issue(j, b):
      pltpu.async_copy(table.at[src_v.at[j]], rbufs[b], gsems[b])
      pltpu.async_copy(dsts.at[wid, j], dst_ib.at[b], dsems[b])

    def step(j, b, fetch_next):
      pltpu.make_async_copy(table.at[src_v.at[j]], rbufs[b], gsems[b]).wait()
      pltpu.make_async_copy(dsts.at[wid, j], dst_ib.at[b], dsems[b]).wait()
      pltpu.sync_copy(rbufs[b], acc.at[dst_ib.at[b]], add=True)
      if fetch_next:
        issue(j + nbuf, b)

    for b in range(nbuf):
      issue(b, b)

    def loop_body(g, _):
      for b in range(nbuf):
        step(g * nbuf + b, b, True)
      return _

    lax.fori_loop(0, cpt // nbuf - 1, loop_body, None)
    for b in range(nbuf):
      step(cpt - nbuf + b, b, False)

    plsc.subcore_barrier()
    pltpu.sync_copy(acc.at[pl.ds(s * rpt, rpt)],
                    out.at[c, pl.ds(s * rpt, rpt)])

  return pl.kernel(
      body,
      out_type=jax.ShapeDtypeStruct((NC, n_pad, d), jnp.float32),
      mesh=_sc_mesh(),
      scratch_types=scratch)


def _sage_dense_body(a0, a1, c0, c1, x, wl, bl, wr, o, *, relu):
  cnt = jnp.maximum(c0[...] + c1[...], 1.0)
  agg = (a0[...] + a1[...]) / cnt
  h = (jnp.dot(agg, wl[...], preferred_element_type=jnp.float32) + bl[...]
       + jnp.dot(x[...], wr[...], preferred_element_type=jnp.float32))
  nrm = jnp.sqrt(jnp.sum(h * h, axis=-1, keepdims=True))
  h = h / jnp.maximum(nrm, 1e-12)
  if relu:
    h = jnp.maximum(h, 0.0)
  o[...] = h


def _final_body(a0, a1, c0, c1, x, wl, bl, wr, wfc, bfc, o):
  cnt = jnp.maximum(c0[...] + c1[...], 1.0)
  agg = (a0[...] + a1[...]) / cnt
  h = (jnp.dot(agg, wl[...], preferred_element_type=jnp.float32) + bl[...]
       + jnp.dot(x[...], wr[...], preferred_element_type=jnp.float32))
  nrm = jnp.sqrt(jnp.sum(h * h, axis=-1, keepdims=True))
  h = h / jnp.maximum(nrm, 1e-12)
  logits = jnp.dot(h, wfc[...], preferred_element_type=jnp.float32) + bfc[...]
  m = jnp.max(logits, axis=-1, keepdims=True)
  e = jnp.exp(logits - m)
  o[...] = e / jnp.sum(e, axis=-1, keepdims=True)


def _dense_call(body, n, d, br, extra_w):
  grid = (n // br,)
  row_spec = pl.BlockSpec((br, d), lambda i: (i, 0))
  cnt_spec = pl.BlockSpec((br, 1), lambda i: (i, 0))
  w_spec = pl.BlockSpec((d, d), lambda i: (0, 0))
  b_spec = pl.BlockSpec((1, d), lambda i: (0, 0))
  in_specs = [row_spec, row_spec, cnt_spec, cnt_spec, row_spec,
              w_spec, b_spec, w_spec] + [w_spec, b_spec] * extra_w
  return pl.pallas_call(
      body,
      grid=grid,
      in_specs=in_specs,
      out_specs=pl.BlockSpec((br, d), lambda i: (i, 0)),
      out_shape=jax.ShapeDtypeStruct((n, d), jnp.float32),
  )


def kernel(feat, edge_index, Wl1, bl1, Wr1, Wl2, bl2, Wr2, Wl3, bl3, Wr3,
           Wfc, bfc):
  n, d = feat.shape
  e = edge_index.shape[1]
  out_dim = Wfc.shape[0]

  ch = 128
  nbuf = 2
  cpt = -(-e // (NW * ch))
  cpt = -(-cpt // nbuf) * nbuf
  cap = NW * cpt * ch
  n_pad = -(-(n + 1) // (NS * 8)) * (NS * 8)

  src = edge_index[0]
  dst = edge_index[1]
  pad = cap - e
  src_p = jnp.concatenate(
      [src, jnp.zeros((pad,), jnp.int32)]).reshape(NW, cpt, ch)
  dst_p = jnp.concatenate(
      [dst, jnp.full((pad,), n, jnp.int32)]).reshape(NW, cpt, ch)

  zeros = jnp.zeros((n_pad, d), jnp.float32)

  agg = _make_sc_agg(n_pad, d, cpt, ch, nbuf)

  br = 1000 if n % 1000 == 0 else n
  dense = _dense_call(functools.partial(_sage_dense_body, relu=True),
                      n, d, br, 0)
  final = _dense_call(_final_body, n, d, br, 1)

  c_parts = agg(jnp.ones((n, d), jnp.float32), src_p, dst_p, zeros)
  c0 = c_parts[0, :n, 0:1]
  c1 = c_parts[1, :n, 0:1]

  a_parts = agg(feat, src_p, dst_p, zeros)
  h1 = dense(a_parts[0, :n], a_parts[1, :n], c0, c1, feat,
             Wl1.T, bl1[None, :], Wr1.T)
  a_parts2 = agg(h1, src_p, dst_p, zeros)
  h2 = dense(a_parts2[0, :n], a_parts2[1, :n], c0, c1, h1,
             Wl2.T, bl2[None, :], Wr2.T)
  a_parts3 = agg(h2, src_p, dst_p, zeros)

  wfc_pad = jnp.zeros((d, d), jnp.float32).at[:, :out_dim].set(Wfc.T)
  bfc_pad = jnp.full((1, d), -1e30, jnp.float32).at[0, :out_dim].set(bfc)
  out = final(a_parts3[0, :n], a_parts3[1, :n], c0, c1, h2,
              Wl3.T, bl3[None, :], Wr3.T, wfc_pad, bfc_pad)
  return out[:, :out_dim]


# speedup vs baseline: 5.2485x; 1.9150x over previous
import functools

import jax
import jax.numpy as jnp
from jax import lax
from jax.experimental import pallas as pl
from jax.experimental.pallas import tpu as pltpu
from jax.experimental.pallas import tpu_sc as plsc

NC = 2
NS = 16
NW = NC * NS
CW = 16


def _sc_mesh():
  return plsc.VectorSubcoreMesh(
      core_axis_name="c", subcore_axis_name="s", num_cores=NC,
      num_subcores=NS)


_SC_PARAMS = pltpu.CompilerParams(use_tc_tiling_on_sc=False,
                                  needs_layout_passes=False)


def _make_sc_agg(n_pad, d, cpt, ch, nbuf):
  rpt = n_pad // NS
  hw = d // 2
  nv = hw // 16

  scratch = (
      [pltpu.VMEM_SHARED((n_pad, d), jnp.float32)] +
      [pltpu.VMEM((cpt, ch), jnp.int32)] +
      [pltpu.VMEM((nbuf, ch), jnp.int32)] +
      [pltpu.VMEM((ch, hw), jnp.int32)] * nbuf +
      [pltpu.VMEM((ch, d), jnp.float32)] +
      [pltpu.SemaphoreType.DMA] * (2 * nbuf))

  def body(table, srcs, dsts, zeros, out, acc, src_v, dst_ib, *rest):
    rbufs = rest[:nbuf]
    frow = rest[nbuf]
    gsems = rest[nbuf + 1:nbuf + 1 + nbuf]
    dsems = rest[nbuf + 1 + nbuf:]

    c = lax.axis_index("c")
    s = lax.axis_index("s")
    wid = c * NS + s

    pltpu.sync_copy(zeros.at[pl.ds(s * rpt, rpt)], acc.at[pl.ds(s * rpt, rpt)])
    pltpu.sync_copy(srcs.at[wid], src_v)
    plsc.subcore_barrier()

    def issue(j, b):
      pltpu.async_copy(table.at[src_v.at[j]], rbufs[b], gsems[b])
      pltpu.async_copy(dsts.at[wid, j], dst_ib.at[b], dsems[b])

    def unpack_rows(b):
      def row_body(r, _):
        for v in range(nv):
          w = rbufs[b][r, pl.ds(v * 16, 16)]
          lo = plsc.bitcast(lax.shift_left(w, 16), jnp.float32)
          hi = plsc.bitcast(w & jnp.int32(-65536), jnp.float32)
          frow[r, pl.ds(v * 16, 16)] = lo
          frow[r, pl.ds(hw + v * 16, 16)] = hi
        return _
      lax.fori_loop(0, ch, row_body, None)

    def step(j, b, fetch_next):
      pltpu.make_async_copy(table.at[src_v.at[j]], rbufs[b], gsems[b]).wait()
      pltpu.make_async_copy(dsts.at[wid, j], dst_ib.at[b], dsems[b]).wait()
      unpack_rows(b)
      if fetch_next:
        issue(j + nbuf, b)
      pltpu.sync_copy(frow, acc.at[dst_ib.at[b]], add=True)

    for b in range(nbuf):
      issue(b, b)

    def loop_body(g, _):
      for b in range(nbuf):
        step(g * nbuf + b, b, True)
      return _

    lax.fori_loop(0, cpt // nbuf - 1, loop_body, None)
    for b in range(nbuf):
      step(cpt - nbuf + b, b, False)

    plsc.subcore_barrier()
    pltpu.sync_copy(acc.at[pl.ds(s * rpt, rpt)],
                    out.at[c, pl.ds(s * rpt, rpt)])

  return pl.kernel(
      body,
      out_type=jax.ShapeDtypeStruct((NC, n_pad, d), jnp.float32),
      mesh=_sc_mesh(),
      compiler_params=_SC_PARAMS,
      scratch_types=scratch)


def _make_sc_cnt(n_pad, cpt, ch):
  rpt = n_pad // NS

  scratch = [
      pltpu.VMEM_SHARED((n_pad, CW), jnp.float32),
      pltpu.VMEM((cpt, ch), jnp.int32),
      pltpu.VMEM((ch, CW), jnp.float32),
  ]

  def body(dsts, zeros_c, ones, cnt_out, cnt_acc, dst_v, ones_v):
    c = lax.axis_index("c")
    s = lax.axis_index("s")
    wid = c * NS + s

    pltpu.sync_copy(zeros_c.at[pl.ds(s * rpt, rpt)],
                    cnt_acc.at[pl.ds(s * rpt, rpt)])
    pltpu.sync_copy(dsts.at[wid], dst_v)
    pltpu.sync_copy(ones, ones_v)
    plsc.subcore_barrier()

    def loop_body(j, _):
      pltpu.sync_copy(ones_v, cnt_acc.at[dst_v.at[j]], add=True)
      return _

    lax.fori_loop(0, cpt, loop_body, None)

    plsc.subcore_barrier()
    pltpu.sync_copy(cnt_acc.at[pl.ds(s * rpt, rpt)],
                    cnt_out.at[c, pl.ds(s * rpt, rpt)])

  return pl.kernel(
      body,
      out_type=jax.ShapeDtypeStruct((NC, n_pad, CW), jnp.float32),
      mesh=_sc_mesh(),
      compiler_params=_SC_PARAMS,
      scratch_types=scratch)


def _sage_dense_body(a0, a1, c0, c1, x, wl, bl, wr, o, *, relu):
  cnt = jnp.maximum(c0[...] + c1[...], 1.0)
  agg = (a0[...] + a1[...]) / cnt
  h = (jnp.dot(agg, wl[...], preferred_element_type=jnp.float32) + bl[...]
       + jnp.dot(x[...], wr[...], preferred_element_type=jnp.float32))
  nrm = jnp.sqrt(jnp.sum(h * h, axis=-1, keepdims=True))
  h = h / jnp.maximum(nrm, 1e-12)
  if relu:
    h = jnp.maximum(h, 0.0)
  o[...] = h


def _final_body(a0, a1, c0, c1, x, wl, bl, wr, wfc, bfc, o):
  cnt = jnp.maximum(c0[...] + c1[...], 1.0)
  agg = (a0[...] + a1[...]) / cnt
  h = (jnp.dot(agg, wl[...], preferred_element_type=jnp.float32) + bl[...]
       + jnp.dot(x[...], wr[...], preferred_element_type=jnp.float32))
  nrm = jnp.sqrt(jnp.sum(h * h, axis=-1, keepdims=True))
  h = h / jnp.maximum(nrm, 1e-12)
  logits = jnp.dot(h, wfc[...], preferred_element_type=jnp.float32) + bfc[...]
  m = jnp.max(logits, axis=-1, keepdims=True)
  e = jnp.exp(logits - m)
  o[...] = e / jnp.sum(e, axis=-1, keepdims=True)


def _dense_call(body, n, d, br, extra_w):
  grid = (n // br,)
  row_spec = pl.BlockSpec((br, d), lambda i: (i, 0))
  cnt_spec = pl.BlockSpec((br, 1), lambda i: (i, 0))
  w_spec = pl.BlockSpec((d, d), lambda i: (0, 0))
  b_spec = pl.BlockSpec((1, d), lambda i: (0, 0))
  in_specs = [row_spec, row_spec, cnt_spec, cnt_spec, row_spec,
              w_spec, b_spec, w_spec] + [w_spec, b_spec] * extra_w
  return pl.pallas_call(
      body,
      grid=grid,
      in_specs=in_specs,
      out_specs=pl.BlockSpec((br, d), lambda i: (i, 0)),
      out_shape=jax.ShapeDtypeStruct((n, d), jnp.float32),
  )


def _pack_bf16(x):
  hw = x.shape[1] // 2
  u = lax.bitcast_convert_type(x.astype(jnp.bfloat16), jnp.uint16)
  w = u[:, :hw].astype(jnp.uint32) | (u[:, hw:].astype(jnp.uint32) << 16)
  return lax.bitcast_convert_type(w, jnp.int32)


def kernel(feat, edge_index, Wl1, bl1, Wr1, Wl2, bl2, Wr2, Wl3, bl3, Wr3,
           Wfc, bfc):
  n, d = feat.shape
  e = edge_index.shape[1]
  out_dim = Wfc.shape[0]

  ch = 128
  nbuf = 2
  cpt = -(-e // (NW * ch))
  cpt = -(-cpt // nbuf) * nbuf
  cap = NW * cpt * ch
  n_pad = -(-(n + 1) // (NS * 8)) * (NS * 8)

  src = edge_index[0]
  dst = edge_index[1]
  pad = cap - e
  src_p = jnp.concatenate(
      [src, jnp.zeros((pad,), jnp.int32)]).reshape(NW, cpt, ch)
  dst_p = jnp.concatenate(
      [dst, jnp.full((pad,), n, jnp.int32)]).reshape(NW, cpt, ch)

  zeros = jnp.zeros((n_pad, d), jnp.float32)
  zeros_c = jnp.zeros((n_pad, CW), jnp.float32)
  ones = jnp.ones((ch, CW), jnp.float32)

  agg = _make_sc_agg(n_pad, d, cpt, ch, nbuf)
  cntk = _make_sc_cnt(n_pad, cpt, ch)

  br = 1000 if n % 1000 == 0 else n
  dense = _dense_call(functools.partial(_sage_dense_body, relu=True),
                      n, d, br, 0)
  final = _dense_call(_final_body, n, d, br, 1)

  c_parts = cntk(dst_p, zeros_c, ones)
  c0 = c_parts[0, :n, 0:1]
  c1 = c_parts[1, :n, 0:1]

  a_parts = agg(_pack_bf16(feat), src_p, dst_p, zeros)
  h1 = dense(a_parts[0, :n], a_parts[1, :n], c0, c1, feat,
             Wl1.T, bl1[None, :], Wr1.T)
  a_parts2 = agg(_pack_bf16(h1), src_p, dst_p, zeros)
  h2 = dense(a_parts2[0, :n], a_parts2[1, :n], c0, c1, h1,
             Wl2.T, bl2[None, :], Wr2.T)
  a_parts3 = agg(_pack_bf16(h2), src_p, dst_p, zeros)

  wfc_pad = jnp.zeros((d, d), jnp.float32).at[:, :out_dim].set(Wfc.T)
  bfc_pad = jnp.full((1, d), -1e30, jnp.float32).at[0, :out_dim].set(bfc)
  out = final(a_parts3[0, :n], a_parts3[1, :n], c0, c1, h2,
              Wl3.T, bl3[None, :], Wr3.T, wfc_pad, bfc_pad)
  return out[:, :out_dim]
